# baseline (speedup 1.0000x reference)
import functools

import jax
import jax.numpy as jnp
import numpy as np
from jax.experimental import pallas as pl
from jax.experimental.pallas import tpu as pltpu

_TR = 16


def _perm_matrix(n_ch: int) -> np.ndarray:
    k = 3 * n_ch * 128
    p = np.zeros((k, k), np.float32)
    for a in range(3):
        for c in range(n_ch):
            j = n_ch * a + c
            for lane in range(128):
                p[j * 128 + lane, n_ch * (3 * lane + a) + c] = 1.0
    return p


def _rpn_body(x_ref, w9_ref, bsh_ref, wh_ref, bh_ref, p2_ref, p4_ref,
              cls_ref, probs_ref, reg_ref, xs, w_bf, wh_bf,
              *, tr, h, w, c, n, nt):
    b = pl.program_id(0)
    i = pl.program_id(1)
    row0 = i * tr
    m = tr * w
    nq = m // 128

    @pl.when((b == 0) & (i == 0))
    def _init():
        w_bf[...] = w9_ref[...].astype(jnp.bfloat16)
        wh_bf[...] = wh_ref[...].astype(jnp.bfloat16)
        xs[:, 0:1, :] = jnp.zeros((tr + 2, 1, c), jnp.bfloat16)
        xs[:, w + 1:w + 2, :] = jnp.zeros((tr + 2, 1, c), jnp.bfloat16)

    @pl.when(i == 0)
    def _top():
        xs[0:1, 1:w + 1, :] = jnp.zeros((1, w, c), jnp.bfloat16)
        xs[1:tr + 2, 1:w + 1, :] = x_ref[0, 0:tr + 1, :, :].astype(jnp.bfloat16)

    @pl.when((i > 0) & (i < nt - 1))
    def _mid():
        xs[0:tr + 2, 1:w + 1, :] = (
            x_ref[0, pl.ds(row0 - 1, tr + 2), :, :].astype(jnp.bfloat16))

    @pl.when(i == nt - 1)
    def _bot():
        xs[tr + 1:tr + 2, 1:w + 1, :] = jnp.zeros((1, w, c), jnp.bfloat16)
        xs[0:tr + 1, 1:w + 1, :] = (
            x_ref[0, pl.ds(row0 - 1, tr + 1), :, :].astype(jnp.bfloat16))

    cols = []
    for dy in range(3):
        for dx in range(3):
            cols.append(xs[dy:dy + tr, dx:dx + w, :].reshape(m, c))
    patch = jnp.concatenate(cols, axis=1)
    acc = jnp.dot(patch, w_bf[...], preferred_element_type=jnp.float32)
    shared = jnp.maximum(acc + bsh_ref[...], 0.0).astype(jnp.bfloat16)
    head = (jnp.dot(shared, wh_bf[...], preferred_element_type=jnp.float32)
            + bh_ref[...])
    cls = head[:, :6]
    rot_l = jnp.concatenate([cls[:, 1:], cls[:, :1]], axis=1)
    rot_r = jnp.concatenate([cls[:, 5:], cls[:, :5]], axis=1)
    lane = jax.lax.broadcasted_iota(jnp.int32, cls.shape, 1)
    swapped = jnp.where(lane % 2 == 0, rot_l, rot_r)
    probs = jax.nn.sigmoid(cls - swapped)
    comb = jnp.concatenate([cls, probs, head[:, 6:18]], axis=1)
    comb_t = comb.astype(jnp.bfloat16).T

    def lanes(j0, nj):
        rows = [comb_t[j:j + 1, :].reshape(1, nq, 128).reshape(nq, 128)
                for j in range(j0, j0 + nj)]
        return jnp.concatenate(rows, axis=1)

    g_cls = jnp.dot(lanes(0, 6), p2_ref[...],
                    preferred_element_type=jnp.float32)
    g_probs = jnp.dot(lanes(6, 6), p2_ref[...],
                      preferred_element_type=jnp.float32)
    g_reg = jnp.dot(lanes(12, 12), p4_ref[...],
                    preferred_element_type=jnp.float32)
    cls_ref[0] = g_cls
    probs_ref[0] = g_probs
    reg_ref[0] = g_reg


@jax.jit
def kernel(inputs, W_shared, b_shared, W_cls, b_cls, W_reg, b_reg):
    B, H, W, C = inputs.shape
    N = W_shared.shape[-1]
    A = W_cls.shape[-1] // 2
    HW = H * W

    w9 = W_shared.reshape(9 * C, N)
    wh = jnp.concatenate([W_cls.reshape(N, 2 * A),
                          W_reg.reshape(N, 4 * A)], axis=1)
    bsh = b_shared.reshape(1, N)
    bh = jnp.concatenate([b_cls, b_reg]).reshape(1, 6 * A)
    p2 = jnp.asarray(_perm_matrix(2), jnp.bfloat16)
    p4 = jnp.asarray(_perm_matrix(4), jnp.bfloat16)

    tr = _TR
    nt = H // tr
    m = tr * W
    nq = m // 128
    body = functools.partial(_rpn_body, tr=tr, h=H, w=W, c=C, n=N, nt=nt)
    cls_f, probs_f, reg_f = pl.pallas_call(
        body,
        grid=(B, nt),
        in_specs=[
            pl.BlockSpec((1, H, W, C), lambda b, i: (b, 0, 0, 0)),
            pl.BlockSpec((9 * C, N), lambda b, i: (0, 0)),
            pl.BlockSpec((1, N), lambda b, i: (0, 0)),
            pl.BlockSpec((N, 6 * A), lambda b, i: (0, 0)),
            pl.BlockSpec((1, 6 * A), lambda b, i: (0, 0)),
            pl.BlockSpec((768, 768), lambda b, i: (0, 0)),
            pl.BlockSpec((1536, 1536), lambda b, i: (0, 0)),
        ],
        out_specs=[
            pl.BlockSpec((1, nq, 768), lambda b, i: (b, i, 0)),
            pl.BlockSpec((1, nq, 768), lambda b, i: (b, i, 0)),
            pl.BlockSpec((1, nq, 1536), lambda b, i: (b, i, 0)),
        ],
        out_shape=[
            jax.ShapeDtypeStruct((B, nt * nq, 768), jnp.float32),
            jax.ShapeDtypeStruct((B, nt * nq, 768), jnp.float32),
            jax.ShapeDtypeStruct((B, nt * nq, 1536), jnp.float32),
        ],
        scratch_shapes=[
            pltpu.VMEM((tr + 2, W + 2, C), jnp.bfloat16),
            pltpu.VMEM((9 * C, N), jnp.bfloat16),
            pltpu.VMEM((N, 6 * A), jnp.bfloat16),
        ],
    )(inputs, w9, bsh, wh, bh, p2, p4)

    rpn_class_logits = cls_f.reshape(B, HW * A, 2)
    rpn_probs = probs_f.reshape(B, HW * A, 2)
    rpn_deltas = reg_f.reshape(B, HW * A, 4)
    return (rpn_class_logits, rpn_probs, rpn_deltas)


# speedup vs baseline: 2.2758x; 2.2758x over previous
import functools

import jax
import jax.numpy as jnp
import numpy as np
from jax.experimental import pallas as pl
from jax.experimental.pallas import tpu as pltpu

_TR = 16


def _perm_matrix(n_ch: int) -> np.ndarray:
    k = 3 * n_ch * 128
    p = np.zeros((k, k), np.float32)
    for a in range(3):
        for c in range(n_ch):
            j = n_ch * a + c
            for lane in range(128):
                p[j * 128 + lane, c * 384 + 3 * lane + a] = 1.0
    return p


def _rpn_body(x_ref, w9_ref, bsh_ref, wh_ref, bh_ref, p2_ref, p4_ref,
              cls_ref, probs_ref, reg_ref, xs, w_bf, wh_bf,
              *, tr, h, w, c, n, nt):
    b = pl.program_id(0)
    i = pl.program_id(1)
    row0 = i * tr
    m = tr * w
    nq = m // 128

    @pl.when((b == 0) & (i == 0))
    def _init():
        w_bf[...] = w9_ref[...].astype(jnp.bfloat16)
        wh_bf[...] = wh_ref[...].astype(jnp.bfloat16)
        xs[:, 0:1, :] = jnp.zeros((tr + 2, 1, c), jnp.bfloat16)
        xs[:, w + 1:w + 2, :] = jnp.zeros((tr + 2, 1, c), jnp.bfloat16)

    @pl.when(i == 0)
    def _top():
        xs[0:1, 1:w + 1, :] = jnp.zeros((1, w, c), jnp.bfloat16)
        xs[1:tr + 2, 1:w + 1, :] = x_ref[0, 0:tr + 1, :, :].astype(jnp.bfloat16)

    @pl.when((i > 0) & (i < nt - 1))
    def _mid():
        xs[0:tr + 2, 1:w + 1, :] = (
            x_ref[0, pl.ds(row0 - 1, tr + 2), :, :].astype(jnp.bfloat16))

    @pl.when(i == nt - 1)
    def _bot():
        xs[tr + 1:tr + 2, 1:w + 1, :] = jnp.zeros((1, w, c), jnp.bfloat16)
        xs[0:tr + 1, 1:w + 1, :] = (
            x_ref[0, pl.ds(row0 - 1, tr + 1), :, :].astype(jnp.bfloat16))

    cols = []
    for dy in range(3):
        for dx in range(3):
            cols.append(xs[dy:dy + tr, dx:dx + w, :].reshape(m, c))
    patch = jnp.concatenate(cols, axis=1)
    acc = jnp.dot(patch, w_bf[...], preferred_element_type=jnp.float32)
    shared = jnp.maximum(acc + bsh_ref[...], 0.0).astype(jnp.bfloat16)
    head = (jnp.dot(shared, wh_bf[...], preferred_element_type=jnp.float32)
            + bh_ref[...])
    cls = head[:, :6]
    rot_l = jnp.concatenate([cls[:, 1:], cls[:, :1]], axis=1)
    rot_r = jnp.concatenate([cls[:, 5:], cls[:, :5]], axis=1)
    lane = jax.lax.broadcasted_iota(jnp.int32, cls.shape, 1)
    swapped = jnp.where(lane % 2 == 0, rot_l, rot_r)
    probs = jax.nn.sigmoid(cls - swapped)
    comb = jnp.concatenate([cls, probs, head[:, 6:18]], axis=1)
    comb_t = comb.astype(jnp.bfloat16).T

    def lanes(j0, nj):
        rows = [comb_t[j:j + 1, :].reshape(1, nq, 128).reshape(nq, 128)
                for j in range(j0, j0 + nj)]
        return jnp.concatenate(rows, axis=1)

    g_cp = jnp.dot(jnp.concatenate([lanes(0, 6), lanes(6, 6)], axis=0),
                   p2_ref[...], preferred_element_type=jnp.float32)
    g_reg = jnp.dot(lanes(12, 12), p4_ref[...],
                    preferred_element_type=jnp.float32)

    def rows_of(g, q0, n_ch):
        g3 = g.reshape(g.shape[0], 3 * n_ch, 128)
        return jnp.concatenate([
            jnp.concatenate(
                [g3[q:q + 1, 3 * c + s:3 * c + s + 1, :].reshape(1, 128)
                 for q in range(q0, q0 + nq) for s in range(3)], axis=1)
            for c in range(n_ch)], axis=0)

    cls_ref[0] = rows_of(g_cp, 0, 2)
    probs_ref[0] = rows_of(g_cp, nq, 2)
    reg_ref[0] = rows_of(g_reg, 0, 4)


@jax.jit
def kernel(inputs, W_shared, b_shared, W_cls, b_cls, W_reg, b_reg):
    B, H, W, C = inputs.shape
    N = W_shared.shape[-1]
    A = W_cls.shape[-1] // 2
    HW = H * W

    w9 = W_shared.reshape(9 * C, N)
    wh = jnp.concatenate([W_cls.reshape(N, 2 * A),
                          W_reg.reshape(N, 4 * A)], axis=1)
    bsh = b_shared.reshape(1, N)
    bh = jnp.concatenate([b_cls, b_reg]).reshape(1, 6 * A)
    p2 = jnp.asarray(_perm_matrix(2), jnp.bfloat16)
    p4 = jnp.asarray(_perm_matrix(4), jnp.bfloat16)

    tr = _TR
    nt = H // tr
    m = tr * W
    nq = m // 128
    body = functools.partial(_rpn_body, tr=tr, h=H, w=W, c=C, n=N, nt=nt)
    cls_f, probs_f, reg_f = pl.pallas_call(
        body,
        grid=(B, nt),
        in_specs=[
            pl.BlockSpec((1, H, W, C), lambda b, i: (b, 0, 0, 0)),
            pl.BlockSpec((9 * C, N), lambda b, i: (0, 0)),
            pl.BlockSpec((1, N), lambda b, i: (0, 0)),
            pl.BlockSpec((N, 6 * A), lambda b, i: (0, 0)),
            pl.BlockSpec((1, 6 * A), lambda b, i: (0, 0)),
            pl.BlockSpec((768, 768), lambda b, i: (0, 0)),
            pl.BlockSpec((1536, 1536), lambda b, i: (0, 0)),
        ],
        out_specs=[
            pl.BlockSpec((1, 2, 3 * m), lambda b, i: (b, 0, i)),
            pl.BlockSpec((1, 2, 3 * m), lambda b, i: (b, 0, i)),
            pl.BlockSpec((1, 4, 3 * m), lambda b, i: (b, 0, i)),
        ],
        out_shape=[
            jax.ShapeDtypeStruct((B, 2, HW * A), jnp.float32),
            jax.ShapeDtypeStruct((B, 2, HW * A), jnp.float32),
            jax.ShapeDtypeStruct((B, 4, HW * A), jnp.float32),
        ],
        scratch_shapes=[
            pltpu.VMEM((tr + 2, W + 2, C), jnp.bfloat16),
            pltpu.VMEM((9 * C, N), jnp.bfloat16),
            pltpu.VMEM((N, 6 * A), jnp.bfloat16),
        ],
    )(inputs, w9, bsh, wh, bh, p2, p4)

    rpn_class_logits = cls_f.transpose(0, 2, 1)
    rpn_probs = probs_f.transpose(0, 2, 1)
    rpn_deltas = reg_f.transpose(0, 2, 1)
    return (rpn_class_logits, rpn_probs, rpn_deltas)


# speedup vs baseline: 2.5675x; 1.1281x over previous
import functools

import jax
import jax.numpy as jnp
import numpy as np
from jax.experimental import pallas as pl
from jax.experimental.pallas import tpu as pltpu

_TR = 32


def _perm_matrix(n_ch: int) -> np.ndarray:
    k = 3 * n_ch * 128
    p = np.zeros((k, k), np.float32)
    for a in range(3):
        for c in range(n_ch):
            j = n_ch * a + c
            for lane in range(128):
                p[j * 128 + lane, c * 384 + 3 * lane + a] = 1.0
    return p


def _rpn_body(x_ref, w9_ref, bsh_ref, wh_ref, bh_ref, p2_ref, p4_ref,
              cls_ref, probs_ref, reg_ref, xs, w_bf, wh_bf,
              *, tr, h, w, c, n, nt):
    b = pl.program_id(0)
    i = pl.program_id(1)
    row0 = i * tr
    m = tr * w
    nq = m // 128

    @pl.when((b == 0) & (i == 0))
    def _init():
        w_bf[...] = w9_ref[...].astype(jnp.bfloat16)
        wh_bf[...] = wh_ref[...].astype(jnp.bfloat16)
        xs[:, 0:1, :] = jnp.zeros((tr + 2, 1, c), jnp.bfloat16)
        xs[:, w + 1:w + 2, :] = jnp.zeros((tr + 2, 1, c), jnp.bfloat16)

    @pl.when(i == 0)
    def _top():
        xs[0:1, 1:w + 1, :] = jnp.zeros((1, w, c), jnp.bfloat16)
        xs[1:tr + 2, 1:w + 1, :] = x_ref[0, 0:tr + 1, :, :].astype(jnp.bfloat16)

    @pl.when((i > 0) & (i < nt - 1))
    def _mid():
        xs[0:tr + 2, 1:w + 1, :] = (
            x_ref[0, pl.ds(row0 - 1, tr + 2), :, :].astype(jnp.bfloat16))

    @pl.when(i == nt - 1)
    def _bot():
        xs[tr + 1:tr + 2, 1:w + 1, :] = jnp.zeros((1, w, c), jnp.bfloat16)
        xs[0:tr + 1, 1:w + 1, :] = (
            x_ref[0, pl.ds(row0 - 1, tr + 1), :, :].astype(jnp.bfloat16))

    sh = [xs[:, dx:dx + w, :] for dx in range(3)]
    cols = [sh[dx][dy:dy + tr].reshape(m, c)
            for dy in range(3) for dx in range(3)]
    patch = jnp.concatenate(cols, axis=1)
    acc = jnp.dot(patch, w_bf[...], preferred_element_type=jnp.float32)
    shared = jnp.maximum(acc + bsh_ref[...], 0.0).astype(jnp.bfloat16)
    head = (jnp.dot(shared, wh_bf[...], preferred_element_type=jnp.float32)
            + bh_ref[...])
    cls = head[:, :6]
    rot_l = jnp.concatenate([cls[:, 1:], cls[:, :1]], axis=1)
    rot_r = jnp.concatenate([cls[:, 5:], cls[:, :5]], axis=1)
    lane = jax.lax.broadcasted_iota(jnp.int32, cls.shape, 1)
    swapped = jnp.where(lane % 2 == 0, rot_l, rot_r)
    probs = jax.nn.sigmoid(cls - swapped)
    comb = jnp.concatenate([cls, probs, head[:, 6:18]], axis=1)
    comb_t = comb.astype(jnp.bfloat16).T

    def lanes(j0, nj):
        rows = [comb_t[j:j + 1, :].reshape(1, nq, 128).reshape(nq, 128)
                for j in range(j0, j0 + nj)]
        return jnp.concatenate(rows, axis=1)

    g_cp = jnp.dot(jnp.concatenate([lanes(0, 6), lanes(6, 6)], axis=0),
                   p2_ref[...], preferred_element_type=jnp.float32)
    g_reg = jnp.dot(lanes(12, 12), p4_ref[...],
                    preferred_element_type=jnp.float32)

    def rows_of(g, q0, n_ch):
        g3 = g.reshape(g.shape[0], 3 * n_ch, 128)
        return jnp.concatenate([
            jnp.concatenate(
                [g3[q:q + 1, 3 * c + s:3 * c + s + 1, :].reshape(1, 128)
                 for q in range(q0, q0 + nq) for s in range(3)], axis=1)
            for c in range(n_ch)], axis=0)

    cls_ref[0] = rows_of(g_cp, 0, 2)
    probs_ref[0] = rows_of(g_cp, nq, 2)
    reg_ref[0] = rows_of(g_reg, 0, 4)


@jax.jit
def kernel(inputs, W_shared, b_shared, W_cls, b_cls, W_reg, b_reg):
    B, H, W, C = inputs.shape
    N = W_shared.shape[-1]
    A = W_cls.shape[-1] // 2
    HW = H * W

    w9 = W_shared.reshape(9 * C, N)
    wh = jnp.concatenate([W_cls.reshape(N, 2 * A),
                          W_reg.reshape(N, 4 * A)], axis=1)
    bsh = b_shared.reshape(1, N)
    bh = jnp.concatenate([b_cls, b_reg]).reshape(1, 6 * A)
    p2 = jnp.asarray(_perm_matrix(2), jnp.bfloat16)
    p4 = jnp.asarray(_perm_matrix(4), jnp.bfloat16)

    tr = _TR
    nt = H // tr
    m = tr * W
    nq = m // 128
    body = functools.partial(_rpn_body, tr=tr, h=H, w=W, c=C, n=N, nt=nt)
    cls_f, probs_f, reg_f = pl.pallas_call(
        body,
        grid=(B, nt),
        in_specs=[
            pl.BlockSpec((1, H, W, C), lambda b, i: (b, 0, 0, 0)),
            pl.BlockSpec((9 * C, N), lambda b, i: (0, 0)),
            pl.BlockSpec((1, N), lambda b, i: (0, 0)),
            pl.BlockSpec((N, 6 * A), lambda b, i: (0, 0)),
            pl.BlockSpec((1, 6 * A), lambda b, i: (0, 0)),
            pl.BlockSpec((768, 768), lambda b, i: (0, 0)),
            pl.BlockSpec((1536, 1536), lambda b, i: (0, 0)),
        ],
        out_specs=[
            pl.BlockSpec((1, 2, 3 * m), lambda b, i: (b, 0, i)),
            pl.BlockSpec((1, 2, 3 * m), lambda b, i: (b, 0, i)),
            pl.BlockSpec((1, 4, 3 * m), lambda b, i: (b, 0, i)),
        ],
        out_shape=[
            jax.ShapeDtypeStruct((B, 2, HW * A), jnp.float32),
            jax.ShapeDtypeStruct((B, 2, HW * A), jnp.float32),
            jax.ShapeDtypeStruct((B, 4, HW * A), jnp.float32),
        ],
        scratch_shapes=[
            pltpu.VMEM((tr + 2, W + 2, C), jnp.bfloat16),
            pltpu.VMEM((9 * C, N), jnp.bfloat16),
            pltpu.VMEM((N, 6 * A), jnp.bfloat16),
        ],
    )(inputs, w9, bsh, wh, bh, p2, p4)

    rpn_class_logits = cls_f.transpose(0, 2, 1)
    rpn_probs = probs_f.transpose(0, 2, 1)
    rpn_deltas = reg_f.transpose(0, 2, 1)
    return (rpn_class_logits, rpn_probs, rpn_deltas)


# speedup vs baseline: 3.3112x; 1.2897x over previous
import functools

import jax
import jax.numpy as jnp
import numpy as np
from jax.experimental import pallas as pl
from jax.experimental.pallas import tpu as pltpu

_TR = 32


def _perm_matrix(n_ch: int) -> np.ndarray:
    k = 3 * n_ch * 128
    p = np.zeros((k, k), np.float32)
    for a in range(3):
        for c in range(n_ch):
            j = n_ch * a + c
            for lane in range(128):
                p[j * 128 + lane, c * 384 + 3 * lane + a] = 1.0
    return p


def _rpn_body(x_ref, w9_ref, bsh_ref, wh_ref, bh_ref, p2_ref,
              cls_ref, probs_ref, reg_ref, xs, w_bf, wh_bf,
              *, tr, h, w, c, n, nt):
    b = pl.program_id(0)
    i = pl.program_id(1)
    row0 = i * tr
    m = tr * w
    nq = m // 128

    @pl.when((b == 0) & (i == 0))
    def _init():
        w_bf[...] = w9_ref[...].astype(jnp.bfloat16)
        wh_bf[...] = wh_ref[...].astype(jnp.bfloat16)
        xs[:, 0:1, :] = jnp.zeros((tr + 2, 1, c), jnp.bfloat16)
        xs[:, w + 1:w + 2, :] = jnp.zeros((tr + 2, 1, c), jnp.bfloat16)

    @pl.when(i == 0)
    def _top():
        xs[0:1, 1:w + 1, :] = jnp.zeros((1, w, c), jnp.bfloat16)
        xs[1:tr + 2, 1:w + 1, :] = x_ref[0, 0:tr + 1, :, :].astype(jnp.bfloat16)

    @pl.when((i > 0) & (i < nt - 1))
    def _mid():
        xs[0:tr + 2, 1:w + 1, :] = (
            x_ref[0, pl.ds(row0 - 1, tr + 2), :, :].astype(jnp.bfloat16))

    @pl.when(i == nt - 1)
    def _bot():
        xs[tr + 1:tr + 2, 1:w + 1, :] = jnp.zeros((1, w, c), jnp.bfloat16)
        xs[0:tr + 1, 1:w + 1, :] = (
            x_ref[0, pl.ds(row0 - 1, tr + 1), :, :].astype(jnp.bfloat16))

    sh = [xs[:, dx:dx + w, :] for dx in range(3)]
    cols = [sh[dx][dy:dy + tr].reshape(m, c)
            for dy in range(3) for dx in range(3)]
    patch = jnp.concatenate(cols, axis=1)
    acc = jnp.dot(patch, w_bf[...], preferred_element_type=jnp.float32)
    shared = jnp.maximum(acc + bsh_ref[...], 0.0).astype(jnp.bfloat16)
    head = (jnp.dot(shared, wh_bf[...], preferred_element_type=jnp.float32)
            + bh_ref[...])
    head_t = head.astype(jnp.bfloat16).T
    cls_t = head_t[0:6]
    swap_t = jnp.concatenate(
        [head_t[j ^ 1:(j ^ 1) + 1] for j in range(6)], axis=0)
    probs_t = jax.nn.sigmoid(cls_t - swap_t)

    def lanes(rows6):
        return jnp.concatenate(
            [r.reshape(1, nq, 128).reshape(nq, 128) for r in rows6], axis=1)

    blocks = [
        [cls_t[j:j + 1] for j in range(6)],
        [probs_t[j:j + 1] for j in range(6)],
        [head_t[6 + 4 * a + e:7 + 4 * a + e] for a in range(3)
         for e in range(2)],
        [head_t[8 + 4 * a + e:9 + 4 * a + e] for a in range(3)
         for e in range(2)],
    ]
    x_all = jnp.concatenate([lanes(b) for b in blocks], axis=0)
    g_all = jnp.dot(x_all, p2_ref[...], preferred_element_type=jnp.float32)

    def rows_of(q0, n_ch):
        g3 = g_all.reshape(4 * nq, 6, 128)
        return jnp.concatenate([
            jnp.concatenate(
                [g3[q:q + 1, 3 * c + s:3 * c + s + 1, :].reshape(1, 128)
                 for q in range(q0, q0 + nq) for s in range(3)], axis=1)
            for c in range(n_ch)], axis=0)

    cls_ref[0] = rows_of(0, 2)
    probs_ref[0] = rows_of(nq, 2)
    reg_ref[0] = jnp.concatenate(
        [rows_of(2 * nq, 2), rows_of(3 * nq, 2)], axis=0)


@jax.jit
def kernel(inputs, W_shared, b_shared, W_cls, b_cls, W_reg, b_reg):
    B, H, W, C = inputs.shape
    N = W_shared.shape[-1]
    A = W_cls.shape[-1] // 2
    HW = H * W

    w9 = W_shared.reshape(9 * C, N)
    wh = jnp.concatenate([W_cls.reshape(N, 2 * A),
                          W_reg.reshape(N, 4 * A)], axis=1)
    bsh = b_shared.reshape(1, N)
    bh = jnp.concatenate([b_cls, b_reg]).reshape(1, 6 * A)
    p2 = jnp.asarray(_perm_matrix(2), jnp.bfloat16)

    tr = _TR
    nt = H // tr
    m = tr * W
    nq = m // 128
    body = functools.partial(_rpn_body, tr=tr, h=H, w=W, c=C, n=N, nt=nt)
    cls_f, probs_f, reg_f = pl.pallas_call(
        body,
        grid=(B, nt),
        in_specs=[
            pl.BlockSpec((1, H, W, C), lambda b, i: (b, 0, 0, 0)),
            pl.BlockSpec((9 * C, N), lambda b, i: (0, 0)),
            pl.BlockSpec((1, N), lambda b, i: (0, 0)),
            pl.BlockSpec((N, 6 * A), lambda b, i: (0, 0)),
            pl.BlockSpec((1, 6 * A), lambda b, i: (0, 0)),
            pl.BlockSpec((768, 768), lambda b, i: (0, 0)),
        ],
        out_specs=[
            pl.BlockSpec((1, 2, 3 * m), lambda b, i: (b, 0, i)),
            pl.BlockSpec((1, 2, 3 * m), lambda b, i: (b, 0, i)),
            pl.BlockSpec((1, 4, 3 * m), lambda b, i: (b, 0, i)),
        ],
        out_shape=[
            jax.ShapeDtypeStruct((B, 2, HW * A), jnp.float32),
            jax.ShapeDtypeStruct((B, 2, HW * A), jnp.float32),
            jax.ShapeDtypeStruct((B, 4, HW * A), jnp.float32),
        ],
        scratch_shapes=[
            pltpu.VMEM((tr + 2, W + 2, C), jnp.bfloat16),
            pltpu.VMEM((9 * C, N), jnp.bfloat16),
            pltpu.VMEM((N, 6 * A), jnp.bfloat16),
        ],
    )(inputs, w9, bsh, wh, bh, p2)

    rpn_class_logits = cls_f.transpose(0, 2, 1)
    rpn_probs = probs_f.transpose(0, 2, 1)
    rpn_deltas = reg_f.transpose(0, 2, 1)
    return (rpn_class_logits, rpn_probs, rpn_deltas)


# speedup vs baseline: 3.4957x; 1.0557x over previous
import functools

import jax
import jax.numpy as jnp
import numpy as np
from jax.experimental import pallas as pl
from jax.experimental.pallas import tpu as pltpu

_TR = 64


def _perm_matrix(n_ch: int) -> np.ndarray:
    k = 3 * n_ch * 128
    p = np.zeros((k, k), np.float32)
    for a in range(3):
        for c in range(n_ch):
            j = n_ch * a + c
            for lane in range(128):
                p[j * 128 + lane, c * 384 + 3 * lane + a] = 1.0
    return p


def _rpn_body(x_ref, w9_ref, bsh_ref, wh_ref, bh_ref, p2_ref,
              cls_ref, probs_ref, reg_ref, xs, w_bf, wh_bf,
              *, tr, h, w, c, n, nt):
    b = pl.program_id(0)
    i = pl.program_id(1)
    row0 = i * tr
    m = tr * w
    nq = m // 128

    @pl.when((b == 0) & (i == 0))
    def _init():
        w_bf[...] = w9_ref[...].astype(jnp.bfloat16)
        wh_bf[...] = wh_ref[...].astype(jnp.bfloat16)
        xs[:, 0:1, :] = jnp.zeros((tr + 2, 1, c), jnp.bfloat16)
        xs[:, w + 1:w + 2, :] = jnp.zeros((tr + 2, 1, c), jnp.bfloat16)

    if nt == 1:
        xs[0:1, 1:w + 1, :] = jnp.zeros((1, w, c), jnp.bfloat16)
        xs[tr + 1:tr + 2, 1:w + 1, :] = jnp.zeros((1, w, c), jnp.bfloat16)
        xs[1:tr + 1, 1:w + 1, :] = x_ref[0].astype(jnp.bfloat16)
    else:
        @pl.when(i == 0)
        def _top():
            xs[0:1, 1:w + 1, :] = jnp.zeros((1, w, c), jnp.bfloat16)
            xs[1:tr + 2, 1:w + 1, :] = (
                x_ref[0, 0:tr + 1, :, :].astype(jnp.bfloat16))

        @pl.when((i > 0) & (i < nt - 1))
        def _mid():
            xs[0:tr + 2, 1:w + 1, :] = (
                x_ref[0, pl.ds(row0 - 1, tr + 2), :, :].astype(jnp.bfloat16))

        @pl.when(i == nt - 1)
        def _bot():
            xs[tr + 1:tr + 2, 1:w + 1, :] = jnp.zeros((1, w, c), jnp.bfloat16)
            xs[0:tr + 1, 1:w + 1, :] = (
                x_ref[0, pl.ds(row0 - 1, tr + 1), :, :].astype(jnp.bfloat16))

    sh = [xs[:, dx:dx + w, :] for dx in range(3)]
    cols = [sh[dx][dy:dy + tr].reshape(m, c)
            for dy in range(3) for dx in range(3)]
    patch = jnp.concatenate(cols, axis=1)
    acc = jnp.dot(patch, w_bf[...], preferred_element_type=jnp.float32)
    shared = jnp.maximum(acc + bsh_ref[...], 0.0).astype(jnp.bfloat16)
    head = (jnp.dot(shared, wh_bf[...], preferred_element_type=jnp.float32)
            + bh_ref[...])
    head_t = head.astype(jnp.bfloat16).T
    cls_t = head_t[0:6]
    swap_t = jnp.concatenate(
        [head_t[j ^ 1:(j ^ 1) + 1] for j in range(6)], axis=0)
    probs_t = jax.nn.sigmoid(cls_t - swap_t)

    def lanes(rows6):
        return jnp.concatenate(
            [r.reshape(1, nq, 128).reshape(nq, 128) for r in rows6], axis=1)

    blocks = [
        [cls_t[j:j + 1] for j in range(6)],
        [probs_t[j:j + 1] for j in range(6)],
        [head_t[6 + 4 * a + e:7 + 4 * a + e] for a in range(3)
         for e in range(2)],
        [head_t[8 + 4 * a + e:9 + 4 * a + e] for a in range(3)
         for e in range(2)],
    ]
    x_all = jnp.concatenate([lanes(b) for b in blocks], axis=0)
    g_all = jnp.dot(x_all, p2_ref[...], preferred_element_type=jnp.float32)

    def rows_of(q0, n_ch):
        g3 = g_all.reshape(4 * nq, 6, 128)
        return jnp.concatenate([
            jnp.concatenate(
                [g3[q:q + 1, 3 * c + s:3 * c + s + 1, :].reshape(1, 128)
                 for q in range(q0, q0 + nq) for s in range(3)], axis=1)
            for c in range(n_ch)], axis=0)

    cls_ref[0] = rows_of(0, 2)
    probs_ref[0] = rows_of(nq, 2)
    reg_ref[0] = jnp.concatenate(
        [rows_of(2 * nq, 2), rows_of(3 * nq, 2)], axis=0)


@jax.jit
def kernel(inputs, W_shared, b_shared, W_cls, b_cls, W_reg, b_reg):
    B, H, W, C = inputs.shape
    N = W_shared.shape[-1]
    A = W_cls.shape[-1] // 2
    HW = H * W

    w9 = W_shared.reshape(9 * C, N)
    wh = jnp.concatenate([W_cls.reshape(N, 2 * A),
                          W_reg.reshape(N, 4 * A)], axis=1)
    bsh = b_shared.reshape(1, N)
    bh = jnp.concatenate([b_cls, b_reg]).reshape(1, 6 * A)
    p2 = jnp.asarray(_perm_matrix(2), jnp.bfloat16)

    tr = _TR
    nt = H // tr
    m = tr * W
    nq = m // 128
    body = functools.partial(_rpn_body, tr=tr, h=H, w=W, c=C, n=N, nt=nt)
    cls_f, probs_f, reg_f = pl.pallas_call(
        body,
        grid=(B, nt),
        in_specs=[
            pl.BlockSpec((1, H, W, C), lambda b, i: (b, 0, 0, 0)),
            pl.BlockSpec((9 * C, N), lambda b, i: (0, 0)),
            pl.BlockSpec((1, N), lambda b, i: (0, 0)),
            pl.BlockSpec((N, 6 * A), lambda b, i: (0, 0)),
            pl.BlockSpec((1, 6 * A), lambda b, i: (0, 0)),
            pl.BlockSpec((768, 768), lambda b, i: (0, 0)),
        ],
        out_specs=[
            pl.BlockSpec((1, 2, 3 * m), lambda b, i: (b, 0, i)),
            pl.BlockSpec((1, 2, 3 * m), lambda b, i: (b, 0, i)),
            pl.BlockSpec((1, 4, 3 * m), lambda b, i: (b, 0, i)),
        ],
        out_shape=[
            jax.ShapeDtypeStruct((B, 2, HW * A), jnp.float32),
            jax.ShapeDtypeStruct((B, 2, HW * A), jnp.float32),
            jax.ShapeDtypeStruct((B, 4, HW * A), jnp.float32),
        ],
        scratch_shapes=[
            pltpu.VMEM((tr + 2, W + 2, C), jnp.bfloat16),
            pltpu.VMEM((9 * C, N), jnp.bfloat16),
            pltpu.VMEM((N, 6 * A), jnp.bfloat16),
        ],
    )(inputs, w9, bsh, wh, bh, p2)

    rpn_class_logits = cls_f.transpose(0, 2, 1)
    rpn_probs = probs_f.transpose(0, 2, 1)
    rpn_deltas = reg_f.transpose(0, 2, 1)
    return (rpn_class_logits, rpn_probs, rpn_deltas)


# speedup vs baseline: 3.5284x; 1.0093x over previous
import functools

import jax
import jax.numpy as jnp
import numpy as np
from jax.experimental import pallas as pl
from jax.experimental.pallas import tpu as pltpu

_TR = 64


def _perm_matrix(n_ch: int) -> np.ndarray:
    k = 3 * n_ch * 128
    p = np.zeros((k, k), np.float32)
    for a in range(3):
        for c in range(n_ch):
            j = n_ch * a + c
            for lane in range(128):
                p[j * 128 + lane, c * 384 + 3 * lane + a] = 1.0
    return p


def _rpn_body(x_ref, w9_ref, bsh_ref, wc_ref, wr_ref, bc_ref, br_ref, p2_ref,
              cls_ref, probs_ref, reg_ref, xs, w_bf, wh_bf, bh_s,
              *, tr, h, w, c, n, nt):
    b = pl.program_id(0)
    i = pl.program_id(1)
    row0 = i * tr
    m = tr * w
    nq = m // 128

    @pl.when((b == 0) & (i == 0))
    def _init():
        w_bf[...] = w9_ref[...].astype(jnp.bfloat16)
        wh_bf[:, 0:6] = wc_ref[...].astype(jnp.bfloat16)
        wh_bf[:, 6:18] = wr_ref[...].astype(jnp.bfloat16)
        bh_s[:, 0:6] = bc_ref[...]
        bh_s[:, 6:18] = br_ref[...]
        xs[:, 0:1, :] = jnp.zeros((tr + 2, 1, c), jnp.bfloat16)
        xs[:, w + 1:w + 2, :] = jnp.zeros((tr + 2, 1, c), jnp.bfloat16)

    if nt == 1:
        xs[0:1, 1:w + 1, :] = jnp.zeros((1, w, c), jnp.bfloat16)
        xs[tr + 1:tr + 2, 1:w + 1, :] = jnp.zeros((1, w, c), jnp.bfloat16)
        xs[1:tr + 1, 1:w + 1, :] = x_ref[0].astype(jnp.bfloat16)
    else:
        @pl.when(i == 0)
        def _top():
            xs[0:1, 1:w + 1, :] = jnp.zeros((1, w, c), jnp.bfloat16)
            xs[1:tr + 2, 1:w + 1, :] = (
                x_ref[0, 0:tr + 1, :, :].astype(jnp.bfloat16))

        @pl.when((i > 0) & (i < nt - 1))
        def _mid():
            xs[0:tr + 2, 1:w + 1, :] = (
                x_ref[0, pl.ds(row0 - 1, tr + 2), :, :].astype(jnp.bfloat16))

        @pl.when(i == nt - 1)
        def _bot():
            xs[tr + 1:tr + 2, 1:w + 1, :] = jnp.zeros((1, w, c), jnp.bfloat16)
            xs[0:tr + 1, 1:w + 1, :] = (
                x_ref[0, pl.ds(row0 - 1, tr + 1), :, :].astype(jnp.bfloat16))

    sh = [xs[:, dx:dx + w, :] for dx in range(3)]
    cols = [sh[dx][dy:dy + tr].reshape(m, c)
            for dy in range(3) for dx in range(3)]
    patch = jnp.concatenate(cols, axis=1)
    acc = jnp.dot(patch, w_bf[...], preferred_element_type=jnp.float32)
    shared = jnp.maximum(acc + bsh_ref[...], 0.0).astype(jnp.bfloat16)
    head = (jnp.dot(shared, wh_bf[...], preferred_element_type=jnp.float32)
            + bh_s[...])
    head_t = head.astype(jnp.bfloat16).T
    cls_t = head_t[0:6]
    swap_t = jnp.concatenate(
        [head_t[j ^ 1:(j ^ 1) + 1] for j in range(6)], axis=0)
    probs_t = jax.nn.sigmoid(cls_t - swap_t)

    def lanes(rows6):
        return jnp.concatenate(
            [r.reshape(1, nq, 128).reshape(nq, 128) for r in rows6], axis=1)

    blocks = [
        [cls_t[j:j + 1] for j in range(6)],
        [probs_t[j:j + 1] for j in range(6)],
        [head_t[6 + 4 * a + e:7 + 4 * a + e] for a in range(3)
         for e in range(2)],
        [head_t[8 + 4 * a + e:9 + 4 * a + e] for a in range(3)
         for e in range(2)],
    ]
    x_all = jnp.concatenate([lanes(b) for b in blocks], axis=0)
    g_all = jnp.dot(x_all, p2_ref[...], preferred_element_type=jnp.float32)

    def rows_of(q0, n_ch):
        g3 = g_all.reshape(4 * nq, 6, 128)
        return jnp.concatenate([
            jnp.concatenate(
                [g3[q:q + 1, 3 * c + s:3 * c + s + 1, :].reshape(1, 128)
                 for q in range(q0, q0 + nq) for s in range(3)], axis=1)
            for c in range(n_ch)], axis=0)

    cls_ref[0] = rows_of(0, 2)
    probs_ref[0] = rows_of(nq, 2)
    reg_ref[0] = jnp.concatenate(
        [rows_of(2 * nq, 2), rows_of(3 * nq, 2)], axis=0)


@jax.jit
def kernel(inputs, W_shared, b_shared, W_cls, b_cls, W_reg, b_reg):
    B, H, W, C = inputs.shape
    N = W_shared.shape[-1]
    A = W_cls.shape[-1] // 2
    HW = H * W

    w9 = W_shared.reshape(9 * C, N)
    wc = W_cls.reshape(N, 2 * A)
    wr = W_reg.reshape(N, 4 * A)
    bsh = b_shared.reshape(1, N)
    bc = b_cls.reshape(1, 2 * A)
    br = b_reg.reshape(1, 4 * A)
    p2 = jnp.asarray(_perm_matrix(2), jnp.bfloat16)

    tr = _TR
    nt = H // tr
    m = tr * W
    nq = m // 128
    body = functools.partial(_rpn_body, tr=tr, h=H, w=W, c=C, n=N, nt=nt)
    cls_f, probs_f, reg_f = pl.pallas_call(
        body,
        grid=(B, nt),
        in_specs=[
            pl.BlockSpec((1, H, W, C), lambda b, i: (b, 0, 0, 0)),
            pl.BlockSpec((9 * C, N), lambda b, i: (0, 0)),
            pl.BlockSpec((1, N), lambda b, i: (0, 0)),
            pl.BlockSpec((N, 2 * A), lambda b, i: (0, 0)),
            pl.BlockSpec((N, 4 * A), lambda b, i: (0, 0)),
            pl.BlockSpec((1, 2 * A), lambda b, i: (0, 0)),
            pl.BlockSpec((1, 4 * A), lambda b, i: (0, 0)),
            pl.BlockSpec((768, 768), lambda b, i: (0, 0)),
        ],
        out_specs=[
            pl.BlockSpec((1, 2, 3 * m), lambda b, i: (b, 0, i)),
            pl.BlockSpec((1, 2, 3 * m), lambda b, i: (b, 0, i)),
            pl.BlockSpec((1, 4, 3 * m), lambda b, i: (b, 0, i)),
        ],
        out_shape=[
            jax.ShapeDtypeStruct((B, 2, HW * A), jnp.float32),
            jax.ShapeDtypeStruct((B, 2, HW * A), jnp.float32),
            jax.ShapeDtypeStruct((B, 4, HW * A), jnp.float32),
        ],
        scratch_shapes=[
            pltpu.VMEM((tr + 2, W + 2, C), jnp.bfloat16),
            pltpu.VMEM((9 * C, N), jnp.bfloat16),
            pltpu.VMEM((N, 6 * A), jnp.bfloat16),
            pltpu.VMEM((1, 6 * A), jnp.float32),
        ],
    )(inputs, w9, bsh, wc, wr, bc, br, p2)

    rpn_class_logits = cls_f.transpose(0, 2, 1)
    rpn_probs = probs_f.transpose(0, 2, 1)
    rpn_deltas = reg_f.transpose(0, 2, 1)
    return (rpn_class_logits, rpn_probs, rpn_deltas)
